# baseline (speedup 1.0000x reference)
import functools

import jax
import jax.numpy as jnp
import numpy as np
from jax import lax
from jax.experimental import pallas as pl
from jax.experimental.pallas import tpu as pltpu
from jax.experimental.pallas import tpu_sc as plsc

_B = 8
_N = 4096
_S = 512
_K = 32
_SB = 128
_D = 128
_RAD2 = np.float32(0.2 ** 2)
_NW = 32
_CH = 128


def _fps_body(xs_ref, ys_ref, zs_ref, cx_ref, cy_ref, cz_ref):
    X = xs_ref[:]
    Y = ys_ref[:]
    Z = zs_ref[:]
    lane = lax.broadcasted_iota(jnp.int32, (_B, _N), 1)
    lane_s = lax.broadcasted_iota(jnp.int32, (_B, _S), 1)

    def step(i, carry):
        dist, f, ax, ay, az = carry
        m = lane == f
        cx = jnp.sum(jnp.where(m, X, 0.0), axis=1, keepdims=True)
        cy = jnp.sum(jnp.where(m, Y, 0.0), axis=1, keepdims=True)
        cz = jnp.sum(jnp.where(m, Z, 0.0), axis=1, keepdims=True)
        sel = lane_s == i
        ax = jnp.where(sel, cx, ax)
        ay = jnp.where(sel, cy, ay)
        az = jnp.where(sel, cz, az)
        dx = X - cx
        dy = Y - cy
        dz = Z - cz
        d = (dx * dx + dy * dy) + dz * dz
        dist = jnp.minimum(dist, d)
        mx = jnp.max(dist, axis=1, keepdims=True)
        f2 = jnp.min(jnp.where(dist == mx, lane, _N), axis=1, keepdims=True)
        return dist, f2, ax, ay, az

    dist0 = jnp.full((_B, _N), 1e10, jnp.float32)
    f0 = jnp.zeros((_B, 1), jnp.int32)
    z_s = jnp.zeros((_B, _S), jnp.float32)
    _, _, ax, ay, az = lax.fori_loop(0, _S, step, (dist0, f0, z_s, z_s, z_s))
    cx_ref[:] = ax
    cy_ref[:] = ay
    cz_ref[:] = az


def _fps_call(xs, ys, zs):
    out = jax.ShapeDtypeStruct((_B, _S), jnp.float32)
    return pl.pallas_call(
        _fps_body,
        out_shape=(out, out, out),
    )(xs, ys, zs)


def _seltc_body(nx_ref, xt_ref, gidx_ref, dm_ref):
    i = pl.program_id(0)
    nx = nx_ref[0]
    xt = xt_ref[0]
    mm = lax.dot_general(nx, xt, (((1,), (0,)), ((), ())),
                         preferred_element_type=jnp.float32)
    sx = nx[:, 0:1]
    sy = nx[:, 1:2]
    sz = nx[:, 2:3]
    sn = (sx * sx + sy * sy) + sz * sz
    dx = xt[0:1, :]
    dy = xt[1:2, :]
    dz = xt[2:3, :]
    dn = (dx * dx + dy * dy) + dz * dz
    d2 = (-2.0 * mm + sn) + dn
    dm_ref[:] = jnp.where(d2 <= _RAD2, d2, jnp.inf)

    ng = _SB // 8
    lane = lax.broadcasted_iota(jnp.int32, (8, _N), 1)
    kio = lax.broadcasted_iota(jnp.int32, (8, _K), 1)

    def kstep(k, carry):
        aidx, aval = carry
        naidx = []
        naval = []
        for g in range(ng):
            rows = dm_ref[g * 8:(g + 1) * 8, :]
            v, c = rows, lane
            w = _N
            while w > 128:
                h = w // 2
                le = v[:, :h] <= v[:, h:]
                v = jnp.where(le, v[:, :h], v[:, h:])
                c = jnp.where(le, c[:, :h], c[:, h:])
                w = h
            mn = jnp.min(v, axis=1, keepdims=True)
            sel = jnp.min(jnp.where(v == mn, c, _N), axis=1, keepdims=True)
            hit = kio == k
            naidx.append(jnp.where(hit, sel, aidx[g]))
            naval.append(jnp.where(hit, mn, aval[g]))
            dm_ref[g * 8:(g + 1) * 8, :] = jnp.where(lane == sel, jnp.inf,
                                                     rows)
        return naidx, naval

    z_i = [jnp.zeros((8, _K), jnp.int32)] * ng
    z_f = [jnp.zeros((8, _K), jnp.float32)] * ng
    aidx, aval = lax.fori_loop(0, _K, kstep, (z_i, z_f))
    base = (i // (_S // _SB)) * _N
    for g in range(ng):
        first = aidx[g][:, 0:1]
        gidx_ref[0, g * 8:(g + 1) * 8, :] = (
            jnp.where(aval[g] > _RAD2, first, aidx[g]) + base)


def _seltc_call(nx_pad, xt):
    nsb = _S // _SB
    return pl.pallas_call(
        _seltc_body,
        grid=(_B * nsb,),
        in_specs=[
            pl.BlockSpec((1, _SB, 8), lambda i: (i // 4, i % 4, 0)),
            pl.BlockSpec((1, 8, _N), lambda i: (i // 4, 0, 0)),
        ],
        out_specs=pl.BlockSpec((1, _SB, _K), lambda i: (i // 4, i % 4, 0)),
        out_shape=jax.ShapeDtypeStruct((_B, _S, _K), jnp.int32),
        scratch_shapes=[pltpu.VMEM((_SB, _N), jnp.float32)],
    )(nx_pad, xt)


def _gather_call(table, idx2):
    nchunk = idx2.shape[0]
    per_w = nchunk // _NW
    mesh = plsc.VectorSubcoreMesh(core_axis_name="c", subcore_axis_name="s")

    @functools.partial(
        pl.kernel,
        mesh=mesh,
        out_type=jax.ShapeDtypeStruct((nchunk, _CH, _D), jnp.float32),
        scratch_types=[
            pltpu.VMEM((_CH,), jnp.int32),
            pltpu.VMEM((_CH,), jnp.int32),
            pltpu.VMEM((_CH, _D), jnp.float32),
            pltpu.VMEM((_CH, _D), jnp.float32),
            pltpu.SemaphoreType.DMA,
            pltpu.SemaphoreType.DMA,
        ],
    )
    def k(table_hbm, idx_hbm, out_hbm, idx0, idx1, rows0, rows1, sem0, sem1):
        c = lax.axis_index("c")
        s = lax.axis_index("s")
        base = (s * 2 + c) * per_w

        pltpu.sync_copy(idx_hbm.at[base], idx0)
        pltpu.async_copy(table_hbm.at[idx0], rows0, sem0)

        def body(j, carry):
            e = base + 2 * j
            pltpu.sync_copy(idx_hbm.at[e + 1], idx1)
            pltpu.async_copy(table_hbm.at[idx1], rows1, sem1)
            pltpu.make_async_copy(table_hbm.at[idx0], rows0, sem0).wait()
            pltpu.sync_copy(rows0, out_hbm.at[e])
            pltpu.sync_copy(idx_hbm.at[e + 2], idx0)
            pltpu.async_copy(table_hbm.at[idx0], rows0, sem0)
            pltpu.make_async_copy(table_hbm.at[idx1], rows1, sem1).wait()
            pltpu.sync_copy(rows1, out_hbm.at[e + 1])
            return carry

        lax.fori_loop(0, per_w // 2 - 1, body, 0)
        e = base + per_w - 2
        pltpu.sync_copy(idx_hbm.at[e + 1], idx1)
        pltpu.async_copy(table_hbm.at[idx1], rows1, sem1)
        pltpu.make_async_copy(table_hbm.at[idx0], rows0, sem0).wait()
        pltpu.sync_copy(rows0, out_hbm.at[e])
        pltpu.make_async_copy(table_hbm.at[idx1], rows1, sem1).wait()
        pltpu.sync_copy(rows1, out_hbm.at[e + 1])

    return k(table, idx2)


def _mlp_body(g_ref, c_ref, w1_ref, b1_ref, w2_ref, b2_ref, a_ref, o_ref):
    g = g_ref[0]
    cpad = c_ref[0]
    feat = g - cpad[:, None, :]
    feat2 = feat.reshape(_SB * _K, _D)
    h = lax.dot_general(feat2, w1_ref[:], (((1,), (0,)), ((), ())),
                        preferred_element_type=jnp.float32) + b1_ref[:]
    h = jnp.maximum(h, 0.0)
    h = lax.dot_general(h, w2_ref[:], (((1,), (0,)), ((), ())),
                        preferred_element_type=jnp.float32) + b2_ref[:]
    h = jnp.maximum(h, 0.0)
    logit = jnp.sum(h * a_ref[:], axis=1).reshape(_SB, _K)
    m = jnp.max(logit, axis=1, keepdims=True)
    e = jnp.exp(logit - m)
    att = e / jnp.sum(e, axis=1, keepdims=True)
    h3 = h.reshape(_SB, _K, 128)
    o_ref[0] = jnp.sum(h3 * att[:, :, None], axis=1)


def _mlp_call(gathered, cpad, W1p, b1r, W2, b2r, ar):
    grid = (_B, _S // _SB)
    full = lambda shape: pl.BlockSpec(shape, lambda b, s: tuple(0 for _ in shape))
    return pl.pallas_call(
        _mlp_body,
        grid=grid,
        in_specs=[
            pl.BlockSpec((1, _SB, _K, _D), lambda b, s: (b, s, 0, 0)),
            pl.BlockSpec((1, _SB, _D), lambda b, s: (b, s, 0)),
            full((_D, 128)),
            full((1, 128)),
            full((128, 128)),
            full((1, 128)),
            full((1, 128)),
        ],
        out_specs=pl.BlockSpec((1, _SB, 128), lambda b, s: (b, s, 0)),
        out_shape=jax.ShapeDtypeStruct((_B, _S, 128), jnp.float32),
    )(gathered, cpad, W1p, b1r, W2, b2r, ar)


def kernel(xyz, points, W1, b1, W2, b2, a):
    xs = xyz[..., 0]
    ys = xyz[..., 1]
    zs = xyz[..., 2]
    cx, cy, cz = _fps_call(xs, ys, zs)
    nx = jnp.stack([cx, cy, cz], axis=-1)
    nx_pad = jnp.concatenate(
        [nx, jnp.zeros((_B, _S, 5), jnp.float32)], axis=-1)
    xt = jnp.concatenate(
        [jnp.stack([xs, ys, zs], axis=1),
         jnp.zeros((_B, 5, _N), jnp.float32)], axis=1)
    gidx = _seltc_call(nx_pad, xt)
    table = jnp.concatenate(
        [xyz, points, jnp.zeros((_B, _N, _D - 67), jnp.float32)],
        axis=-1).reshape(_B * _N, _D)
    idx2 = gidx.reshape(-1, _CH)
    gathered = _gather_call(table, idx2).reshape(_B, _S, _K, _D)

    cpad = jnp.concatenate(
        [nx, jnp.zeros((_B, _S, _D - 3), jnp.float32)], axis=-1)
    W1p = jnp.concatenate(
        [W1, jnp.zeros((_D - 67, 128), jnp.float32)], axis=0)
    return _mlp_call(gathered, cpad, W1p, b1.reshape(1, 128), W2,
                     b2.reshape(1, 128), a.reshape(1, 128))


# speedup vs baseline: 1.0791x; 1.0791x over previous
import functools

import jax
import jax.numpy as jnp
import numpy as np
from jax import lax
from jax.experimental import pallas as pl
from jax.experimental.pallas import tpu as pltpu
from jax.experimental.pallas import tpu_sc as plsc

_B = 8
_N = 4096
_S = 512
_K = 32
_SB = 128
_D = 128
_RAD2 = np.float32(0.2 ** 2)
_NW = 32
_CH = 128


def _fps_body(xs_ref, ys_ref, zs_ref, cx_ref, cy_ref, cz_ref):
    X = xs_ref[:]
    Y = ys_ref[:]
    Z = zs_ref[:]
    lane = lax.broadcasted_iota(jnp.int32, (_B, _N), 1)
    lane_s = lax.broadcasted_iota(jnp.int32, (_B, _S), 1)

    def step(i, carry):
        dist, f, ax, ay, az = carry
        m = lane == f
        cx = jnp.sum(jnp.where(m, X, 0.0), axis=1, keepdims=True)
        cy = jnp.sum(jnp.where(m, Y, 0.0), axis=1, keepdims=True)
        cz = jnp.sum(jnp.where(m, Z, 0.0), axis=1, keepdims=True)
        sel = lane_s == i
        ax = jnp.where(sel, cx, ax)
        ay = jnp.where(sel, cy, ay)
        az = jnp.where(sel, cz, az)
        dx = X - cx
        dy = Y - cy
        dz = Z - cz
        d = (dx * dx + dy * dy) + dz * dz
        dist = jnp.minimum(dist, d)
        mx = jnp.max(dist, axis=1, keepdims=True)
        f2 = jnp.min(jnp.where(dist == mx, lane, _N), axis=1, keepdims=True)
        return dist, f2, ax, ay, az

    dist0 = jnp.full((_B, _N), 1e10, jnp.float32)
    f0 = jnp.zeros((_B, 1), jnp.int32)
    z_s = jnp.zeros((_B, _S), jnp.float32)
    _, _, ax, ay, az = lax.fori_loop(0, _S, step, (dist0, f0, z_s, z_s, z_s))
    cx_ref[:] = ax
    cy_ref[:] = ay
    cz_ref[:] = az


def _fps_call(xs, ys, zs):
    out = jax.ShapeDtypeStruct((_B, _S), jnp.float32)
    return pl.pallas_call(
        _fps_body,
        out_shape=(out, out, out),
    )(xs, ys, zs)


def _seltc_body(nx_ref, xt_ref, gidx_ref, dm_ref):
    i = pl.program_id(0)
    nx = nx_ref[0]
    xt = xt_ref[0]
    mm = lax.dot_general(nx, xt, (((1,), (0,)), ((), ())),
                         preferred_element_type=jnp.float32)
    sx = nx[:, 0:1]
    sy = nx[:, 1:2]
    sz = nx[:, 2:3]
    sn = (sx * sx + sy * sy) + sz * sz
    dx = xt[0:1, :]
    dy = xt[1:2, :]
    dz = xt[2:3, :]
    dn = (dx * dx + dy * dy) + dz * dz
    d2 = (-2.0 * mm + sn) + dn
    dm_ref[:] = jnp.where(d2 <= _RAD2, d2, jnp.inf)

    ng = _SB // 8
    lane = lax.broadcasted_iota(jnp.int32, (8, _N), 1)
    kio = lax.broadcasted_iota(jnp.int32, (8, _K), 1)

    def kstep(k, carry):
        aidx, aval = carry
        naidx = []
        naval = []
        for g in range(ng):
            rows = dm_ref[g * 8:(g + 1) * 8, :]
            mn = jnp.min(rows, axis=1, keepdims=True)
            sel = jnp.min(jnp.where(rows == mn, lane, _N), axis=1,
                          keepdims=True)
            hit = kio == k
            naidx.append(jnp.where(hit, sel, aidx[g]))
            naval.append(jnp.where(hit, mn, aval[g]))
            dm_ref[g * 8:(g + 1) * 8, :] = jnp.where(lane == sel, jnp.inf,
                                                     rows)
        return naidx, naval

    z_i = [jnp.zeros((8, _K), jnp.int32)] * ng
    z_f = [jnp.zeros((8, _K), jnp.float32)] * ng
    aidx, aval = lax.fori_loop(0, _K, kstep, (z_i, z_f))
    base = (i // (_S // _SB)) * _N
    for g in range(ng):
        first = aidx[g][:, 0:1]
        gidx_ref[0, g * 8:(g + 1) * 8, :] = (
            jnp.where(aval[g] > _RAD2, first, aidx[g]) + base)


def _seltc_call(nx_pad, xt):
    nsb = _S // _SB
    return pl.pallas_call(
        _seltc_body,
        grid=(_B * nsb,),
        in_specs=[
            pl.BlockSpec((1, _SB, 8), lambda i: (i // 4, i % 4, 0)),
            pl.BlockSpec((1, 8, _N), lambda i: (i // 4, 0, 0)),
        ],
        out_specs=pl.BlockSpec((1, _SB, _K), lambda i: (i // 4, i % 4, 0)),
        out_shape=jax.ShapeDtypeStruct((_B, _S, _K), jnp.int32),
        scratch_shapes=[pltpu.VMEM((_SB, _N), jnp.float32)],
    )(nx_pad, xt)


def _gather_call(table, idx2):
    nchunk = idx2.shape[0]
    per_w = nchunk // _NW
    mesh = plsc.VectorSubcoreMesh(core_axis_name="c", subcore_axis_name="s")

    @functools.partial(
        pl.kernel,
        mesh=mesh,
        out_type=jax.ShapeDtypeStruct((nchunk, _CH, _D), jnp.float32),
        scratch_types=[
            pltpu.VMEM((_CH,), jnp.int32),
            pltpu.VMEM((_CH,), jnp.int32),
            pltpu.VMEM((_CH, _D), jnp.float32),
            pltpu.VMEM((_CH, _D), jnp.float32),
            pltpu.SemaphoreType.DMA,
            pltpu.SemaphoreType.DMA,
        ],
    )
    def k(table_hbm, idx_hbm, out_hbm, idx0, idx1, rows0, rows1, sem0, sem1):
        c = lax.axis_index("c")
        s = lax.axis_index("s")
        base = (s * 2 + c) * per_w

        pltpu.sync_copy(idx_hbm.at[base], idx0)
        pltpu.async_copy(table_hbm.at[idx0], rows0, sem0)

        def body(j, carry):
            e = base + 2 * j
            pltpu.sync_copy(idx_hbm.at[e + 1], idx1)
            pltpu.async_copy(table_hbm.at[idx1], rows1, sem1)
            pltpu.make_async_copy(table_hbm.at[idx0], rows0, sem0).wait()
            pltpu.sync_copy(rows0, out_hbm.at[e])
            pltpu.sync_copy(idx_hbm.at[e + 2], idx0)
            pltpu.async_copy(table_hbm.at[idx0], rows0, sem0)
            pltpu.make_async_copy(table_hbm.at[idx1], rows1, sem1).wait()
            pltpu.sync_copy(rows1, out_hbm.at[e + 1])
            return carry

        lax.fori_loop(0, per_w // 2 - 1, body, 0)
        e = base + per_w - 2
        pltpu.sync_copy(idx_hbm.at[e + 1], idx1)
        pltpu.async_copy(table_hbm.at[idx1], rows1, sem1)
        pltpu.make_async_copy(table_hbm.at[idx0], rows0, sem0).wait()
        pltpu.sync_copy(rows0, out_hbm.at[e])
        pltpu.make_async_copy(table_hbm.at[idx1], rows1, sem1).wait()
        pltpu.sync_copy(rows1, out_hbm.at[e + 1])

    return k(table, idx2)


def _mlp_body(g_ref, c_ref, w1_ref, b1_ref, w2_ref, b2_ref, a_ref, o_ref):
    g = g_ref[0]
    cpad = c_ref[0]
    feat = g - cpad[:, None, :]
    feat2 = feat.reshape(_SB * _K, _D)
    h = lax.dot_general(feat2, w1_ref[:], (((1,), (0,)), ((), ())),
                        preferred_element_type=jnp.float32) + b1_ref[:]
    h = jnp.maximum(h, 0.0)
    h = lax.dot_general(h, w2_ref[:], (((1,), (0,)), ((), ())),
                        preferred_element_type=jnp.float32) + b2_ref[:]
    h = jnp.maximum(h, 0.0)
    logit = jnp.sum(h * a_ref[:], axis=1).reshape(_SB, _K)
    m = jnp.max(logit, axis=1, keepdims=True)
    e = jnp.exp(logit - m)
    att = e / jnp.sum(e, axis=1, keepdims=True)
    h3 = h.reshape(_SB, _K, 128)
    o_ref[0] = jnp.sum(h3 * att[:, :, None], axis=1)


def _mlp_call(gathered, cpad, W1p, b1r, W2, b2r, ar):
    grid = (_B, _S // _SB)
    full = lambda shape: pl.BlockSpec(shape, lambda b, s: tuple(0 for _ in shape))
    return pl.pallas_call(
        _mlp_body,
        grid=grid,
        in_specs=[
            pl.BlockSpec((1, _SB, _K, _D), lambda b, s: (b, s, 0, 0)),
            pl.BlockSpec((1, _SB, _D), lambda b, s: (b, s, 0)),
            full((_D, 128)),
            full((1, 128)),
            full((128, 128)),
            full((1, 128)),
            full((1, 128)),
        ],
        out_specs=pl.BlockSpec((1, _SB, 128), lambda b, s: (b, s, 0)),
        out_shape=jax.ShapeDtypeStruct((_B, _S, 128), jnp.float32),
    )(gathered, cpad, W1p, b1r, W2, b2r, ar)


def kernel(xyz, points, W1, b1, W2, b2, a):
    xs = xyz[..., 0]
    ys = xyz[..., 1]
    zs = xyz[..., 2]
    cx, cy, cz = _fps_call(xs, ys, zs)
    nx = jnp.stack([cx, cy, cz], axis=-1)
    nx_pad = jnp.concatenate(
        [nx, jnp.zeros((_B, _S, 5), jnp.float32)], axis=-1)
    xt = jnp.concatenate(
        [jnp.stack([xs, ys, zs], axis=1),
         jnp.zeros((_B, 5, _N), jnp.float32)], axis=1)
    gidx = _seltc_call(nx_pad, xt)
    table = jnp.concatenate(
        [xyz, points, jnp.zeros((_B, _N, _D - 67), jnp.float32)],
        axis=-1).reshape(_B * _N, _D)
    idx2 = gidx.reshape(-1, _CH)
    gathered = _gather_call(table, idx2).reshape(_B, _S, _K, _D)

    cpad = jnp.concatenate(
        [nx, jnp.zeros((_B, _S, _D - 3), jnp.float32)], axis=-1)
    W1p = jnp.concatenate(
        [W1, jnp.zeros((_D - 67, 128), jnp.float32)], axis=0)
    return _mlp_call(gathered, cpad, W1p, b1.reshape(1, 128), W2,
                     b2.reshape(1, 128), a.reshape(1, 128))
